# baseline (speedup 1.0000x reference)
import jax
import jax.numpy as jnp
from jax.experimental import pallas as pl
from jax.experimental.pallas import tpu as pltpu

_B, _S, _N, _D = 32, 48, 64, 64
_QD = 5
_QDP = 8
_BC = 16
_HI = jax.lax.Precision.HIGHEST
_MM = jax.lax.Precision.DEFAULT


def _dot(a, b):
    return jax.lax.dot_general(a, b, (((1,), (0,)), ((), ())),
                               precision=_MM,
                               preferred_element_type=jnp.float32)


def _precompute_body(plm_ref, psW1_ref, psb1_ref, psW2_ref, psb2_ref,
                     pgW1_ref, pgb1_ref, pgW2_ref, pgb2_ref, bcat_ref, len_ref,
                     adj_ref, vv_ref, beff_ref, emask_ref):
    plm = plm_ref[...]
    h1 = jnp.maximum(jnp.dot(plm, psW1_ref[...], precision=_HI,
                             preferred_element_type=jnp.float32) + psb1_ref[...], 0.0)
    vv = jnp.dot(h1, psW2_ref[...], precision=_HI,
                 preferred_element_type=jnp.float32) + psb2_ref[...]
    h2 = jnp.maximum(jnp.dot(plm, pgW1_ref[...], precision=_HI,
                             preferred_element_type=jnp.float32) + pgb1_ref[...], 0.0)
    pg = jnp.dot(h2, pgW2_ref[...], precision=_HI,
                 preferred_element_type=jnp.float32) + pgb2_ref[...]
    nrm = jnp.sqrt(jnp.sum(pg * pg, axis=1, keepdims=True))
    nor = pg / jnp.maximum(nrm, 1e-12)
    logits = jax.lax.dot_general(nor, nor, (((1,), (1,)), ((), ())),
                                 precision=_HI, preferred_element_type=jnp.float32)
    mx = jnp.max(logits, axis=1, keepdims=True)
    e = jnp.exp(logits - mx)
    adj_ref[...] = e / jnp.sum(e, axis=1, keepdims=True)
    vv_ref[...] = vv
    beff_ref[...] = jnp.dot(vv, bcat_ref[...], precision=_HI,
                            preferred_element_type=jnp.float32)
    s_iota = jax.lax.broadcasted_iota(jnp.int32, (_B, _S), 1)
    emask_ref[...] = (s_iota == (len_ref[...] - 1)).astype(jnp.float32)


def _recurrence_body(obs_ref, maskf_ref, avg_ref, emask_ref, adj_ref, rw_ref,
                     vv_ref, wbig_ru_ref, wrar_ru_ref, wbig_c_ref, wrar_c_ref,
                     beff_ref, out_ref):
    N, D, BC = _N, _D, _BC
    M = BC * N
    mask_all = maskf_ref[...]
    vto3 = jnp.sum(mask_all, axis=1, keepdims=True)
    denom3 = vto3 + 1.0
    rw = rw_ref[...]
    row_i = jax.lax.broadcasted_iota(jnp.int32, (N, N), 0)
    col_i = jax.lax.broadcasted_iota(jnp.int32, (N, N), 1)
    eye = (row_i == col_i).astype(jnp.float32)
    adj_noI = (adj_ref[...] * (1.0 - eye))[None]
    adj_rw = adj_noI * rw
    eye3 = eye[None]
    eye3b = jnp.broadcast_to(eye3, (BC, N, N))
    vv = vv_ref[...]
    vv_t = jnp.concatenate([vv] * BC, axis=0)
    beff = beff_ref[...]
    beff_t = jnp.concatenate([beff] * BC, axis=0)
    wbig_ru = wbig_ru_ref[...]
    wbig_c = wbig_c_ref[...]
    vvw_ru = _dot(vv_t, wrar_ru_ref[...])
    vvw_c = _dot(vv_t, wrar_c_ref[...])
    beff_ru = beff_t[:, :2 * D]
    beff_c = beff_t[:, 2 * D:]
    vvC = [vv_t[:, d:d + 1] for d in range(_QD)]

    def expand_qd(T):
        return jnp.concatenate([T * vvC[d] for d in range(_QD)], axis=1)

    def step_fn(s, carry):
        h2, out2 = carry
        avg3 = avg_ref[:, pl.ds(s, 1), :]
        m3 = maskf_ref[:, pl.ds(s, 1), :]
        rar3 = 0.5 * jnp.tanh(avg3 / denom3)
        rar_c3 = jnp.transpose(rar3, (0, 2, 1))
        m_c3 = jnp.transpose(m3, (0, 2, 1))
        amod3 = adj_noI - adj_rw * jnp.abs(rar_c3 - rar3)
        cur_adj3 = amod3 * (m_c3 * m3) + eye3
        obs3 = obs_ref[:, s]
        h3 = h2.reshape(BC, N, D)
        rarpad3 = jnp.pad(rar_c3, ((0, 0), (0, 0), (0, D - 1)))
        xh3 = jnp.concatenate([obs3, h3, rarpad3], axis=2)
        comb_full = jax.lax.dot_general(
            cur_adj3, xh3, (((2,), (1,)), ((0,), (0,))),
            precision=_MM, preferred_element_type=jnp.float32
        ).reshape(M, 3 * D)
        comb2 = comb_full[:, :2 * D]
        comb_rar2 = comb_full[:, 2 * D:2 * D + 1]
        ru = _dot(expand_qd(comb2), wbig_ru) + comb_rar2 * vvw_ru + beff_ru
        r_g = jax.nn.sigmoid(ru[:, :D])
        u_g = jax.nn.sigmoid(ru[:, D:])
        m_col2 = m_c3.reshape(M, 1)
        rar_col2 = rar_c3.reshape(M, 1)
        mB = jnp.broadcast_to(m_col2, (M, D))
        h_res = h2 * (mB * (r_g - 1.0) + 1.0)
        xr2 = jnp.concatenate([obs3.reshape(M, D), h_res], axis=1)
        tc = _dot(expand_qd(xr2), wbig_c) + rar_col2 * vvw_c + beff_c
        cand = jnp.tanh(tc)
        h_upd = h_res + u_g * (cand - h_res)
        h_new = h2 + mB * (h_upd - h2)
        em = emask_ref[:, pl.ds(s, 1), :]
        em2 = jnp.broadcast_to(em, (BC, N, 1)).reshape(M, 1)
        out2 = out2 + h_new * em2
        return h_new, out2

    z = jnp.zeros((M, D), dtype=jnp.float32)
    _, out2 = jax.lax.fori_loop(0, _S, step_fn, (z, z), unroll=4)
    out_ref[...] = out2.reshape(BC, N, D)


def kernel(obs_emb, observed_mask, lengths, avg_interval, var_plm_rep_tensor,
           rarity_W, W_u, b_u, W_r, b_r, W_c, b_c, ps_W1, ps_b1, ps_W2, ps_b2,
           pg_W1, pg_b1, pg_W2, pg_b2):
    B, S, N, D = obs_emb.shape
    QD = W_u.shape[0]
    f32 = jnp.float32

    def big(W):
        W_oh = jnp.concatenate([W[:, :D, :], W[:, D + 1:, :]], axis=1)
        return W_oh.reshape(QD * 2 * D, D)

    wbig_ru = jnp.concatenate([big(W_r), big(W_u)], axis=1)
    wbig_c = big(W_c)
    def rarrow(W):
        return jnp.pad(W[:, D, :], ((0, _QDP - QD), (0, 0)))
    wrar_ru = jnp.concatenate([rarrow(W_r), rarrow(W_u)], axis=1)
    wrar_c = rarrow(W_c)
    bcat = jnp.concatenate([b_r, b_u, b_c], axis=1)
    bcat = jnp.pad(bcat, ((0, _QDP - QD), (0, 0)))
    psW2 = jnp.pad(ps_W2, ((0, 0), (0, _QDP - QD)))
    psb2 = jnp.pad(ps_b2, (0, _QDP - QD)).reshape(1, _QDP)
    psb1 = ps_b1.reshape(1, -1)
    pgb1 = pg_b1.reshape(1, -1)
    pgb2 = pg_b2.reshape(1, -1)
    maskf = observed_mask.astype(f32)

    adjN, vv, beff, emask = pl.pallas_call(
        _precompute_body,
        out_shape=[
            jax.ShapeDtypeStruct((N, N), f32),
            jax.ShapeDtypeStruct((N, _QDP), f32),
            jax.ShapeDtypeStruct((N, 3 * D), f32),
            jax.ShapeDtypeStruct((B, S), f32),
        ],
    )(var_plm_rep_tensor, ps_W1, psb1, psW2, psb2,
      pg_W1, pgb1, pg_W2, pgb2, bcat, lengths)

    BC = _BC
    rep = lambda *shape: pl.BlockSpec(shape, lambda b: (0,) * len(shape))
    out_call = pl.pallas_call(
        _recurrence_body,
        grid=(B // BC,),
        in_specs=[
            pl.BlockSpec((BC, S, N, D), lambda b: (b, 0, 0, 0)),
            pl.BlockSpec((BC, S, N), lambda b: (b, 0, 0)),
            pl.BlockSpec((BC, S, N), lambda b: (b, 0, 0)),
            pl.BlockSpec((BC, S, 1), lambda b: (b, 0, 0)),
            rep(N, N),
            rep(N, N),
            rep(N, _QDP),
            rep(QD * 2 * D, 2 * D),
            rep(_QDP, 2 * D),
            rep(QD * 2 * D, D),
            rep(_QDP, D),
            rep(N, 3 * D),
        ],
        out_specs=pl.BlockSpec((BC, N, D), lambda b: (b, 0, 0)),
        out_shape=jax.ShapeDtypeStruct((B, N, D), f32),
        compiler_params=pltpu.CompilerParams(
            dimension_semantics=("parallel",),
            vmem_limit_bytes=100 * 1024 * 1024),
    )
    out = out_call(obs_emb, maskf, avg_interval, emask.reshape(B, S, 1),
                   adjN, rarity_W, vv, wbig_ru, wrar_ru, wbig_c, wrar_c, beff)
    return out


# speedup vs baseline: 1.0254x; 1.0254x over previous
import jax
import jax.numpy as jnp
from jax.experimental import pallas as pl
from jax.experimental.pallas import tpu as pltpu

_B, _S, _N, _D = 32, 48, 64, 64
_QD = 5
_QDP = 8
_BC = 16
_HI = jax.lax.Precision.HIGHEST
_MM = jax.lax.Precision.DEFAULT


def _dot(a, b):
    return jax.lax.dot_general(a, b, (((1,), (0,)), ((), ())),
                               precision=_MM,
                               preferred_element_type=jnp.float32)


def _precompute_body(plm_ref, psW1_ref, psb1_ref, psW2_ref, psb2_ref,
                     pgW1_ref, pgb1_ref, pgW2_ref, pgb2_ref, bcat_ref, len_ref,
                     adj_ref, vv_ref, beff_ref, emask_ref):
    plm = plm_ref[...]
    h1 = jnp.maximum(jnp.dot(plm, psW1_ref[...], precision=_HI,
                             preferred_element_type=jnp.float32) + psb1_ref[...], 0.0)
    vv = jnp.dot(h1, psW2_ref[...], precision=_HI,
                 preferred_element_type=jnp.float32) + psb2_ref[...]
    h2 = jnp.maximum(jnp.dot(plm, pgW1_ref[...], precision=_HI,
                             preferred_element_type=jnp.float32) + pgb1_ref[...], 0.0)
    pg = jnp.dot(h2, pgW2_ref[...], precision=_HI,
                 preferred_element_type=jnp.float32) + pgb2_ref[...]
    nrm = jnp.sqrt(jnp.sum(pg * pg, axis=1, keepdims=True))
    nor = pg / jnp.maximum(nrm, 1e-12)
    logits = jax.lax.dot_general(nor, nor, (((1,), (1,)), ((), ())),
                                 precision=_HI, preferred_element_type=jnp.float32)
    mx = jnp.max(logits, axis=1, keepdims=True)
    e = jnp.exp(logits - mx)
    adj_ref[...] = e / jnp.sum(e, axis=1, keepdims=True)
    vv_ref[...] = vv
    beff_ref[...] = jnp.dot(vv, bcat_ref[...], precision=_HI,
                            preferred_element_type=jnp.float32)
    s_iota = jax.lax.broadcasted_iota(jnp.int32, (_B, _S), 1)
    emask_ref[...] = (s_iota == (len_ref[...] - 1)).astype(jnp.float32)


def _recurrence_body(obs_ref, maskf_ref, avg_ref, emask_ref, adj_ref, rw_ref,
                     vv_ref, wbig_ru_ref, wrar_ru_ref, wbig_c_ref, wrar_c_ref,
                     beff_ref, out_ref):
    N, D, BC = _N, _D, _BC
    M = BC * N
    mask_all = maskf_ref[...]
    vto3 = jnp.sum(mask_all, axis=1, keepdims=True)
    denom3 = vto3 + 1.0
    rw = rw_ref[...]
    row_i = jax.lax.broadcasted_iota(jnp.int32, (N, N), 0)
    col_i = jax.lax.broadcasted_iota(jnp.int32, (N, N), 1)
    eye = (row_i == col_i).astype(jnp.float32)
    adj_noI = (adj_ref[...] * (1.0 - eye))[None]
    adj_rw = adj_noI * rw
    eye3 = eye[None]
    eye3b = jnp.broadcast_to(eye3, (BC, N, N))
    vv = vv_ref[...]
    vv_t = jnp.concatenate([vv] * BC, axis=0)
    beff = beff_ref[...]
    beff_t = jnp.concatenate([beff] * BC, axis=0)
    wbig_ru = wbig_ru_ref[...]
    wbig_c = wbig_c_ref[...]
    vvw_ru = _dot(vv_t, wrar_ru_ref[...])
    vvw_c = _dot(vv_t, wrar_c_ref[...])
    beff_ru = beff_t[:, :2 * D]
    beff_c = beff_t[:, 2 * D:]
    vvC = [vv_t[:, d:d + 1] for d in range(_QD)]

    def expand_qd(T):
        return jnp.concatenate([T * vvC[d] for d in range(_QD)], axis=1)

    def step_fn(s, carry):
        h2, out2 = carry
        avg3 = avg_ref[:, pl.ds(s, 1), :]
        m3 = maskf_ref[:, pl.ds(s, 1), :]
        rar3 = 0.5 * jnp.tanh(avg3 / denom3)
        rm3 = jnp.concatenate([rar3, m3], axis=1)
        cols = jax.lax.dot_general(
            eye3b, rm3, (((2,), (2,)), ((0,), (0,))),
            precision=_MM, preferred_element_type=jnp.float32)
        rar_c3 = cols[:, :, 0:1]
        m_c3 = cols[:, :, 1:2]
        amod3 = adj_noI - adj_rw * jnp.abs(rar_c3 - rar3)
        cur_adj3 = amod3 * (m_c3 * m3) + eye3
        obs3 = obs_ref[:, s]
        h3 = h2.reshape(BC, N, D)
        rarpad3 = jnp.pad(rar_c3, ((0, 0), (0, 0), (0, D - 1)))
        xh3 = jnp.concatenate([obs3, h3, rarpad3], axis=2)
        comb_full = jax.lax.dot_general(
            cur_adj3, xh3, (((2,), (1,)), ((0,), (0,))),
            precision=_MM, preferred_element_type=jnp.float32
        ).reshape(M, 3 * D)
        comb2 = comb_full[:, :2 * D]
        comb_rar2 = comb_full[:, 2 * D:2 * D + 1]
        ru = _dot(expand_qd(comb2), wbig_ru) + comb_rar2 * vvw_ru + beff_ru
        r_g = jax.nn.sigmoid(ru[:, :D])
        u_g = jax.nn.sigmoid(ru[:, D:])
        m_col2 = m_c3.reshape(M, 1)
        rar_col2 = rar_c3.reshape(M, 1)
        mB = jnp.broadcast_to(m_col2, (M, D))
        h_res = h2 * (mB * (r_g - 1.0) + 1.0)
        xr2 = jnp.concatenate([obs3.reshape(M, D), h_res], axis=1)
        tc = _dot(expand_qd(xr2), wbig_c) + rar_col2 * vvw_c + beff_c
        cand = jnp.tanh(tc)
        h_upd = h_res + u_g * (cand - h_res)
        h_new = h2 + mB * (h_upd - h2)
        em = emask_ref[:, pl.ds(s, 1), :]
        em2 = jnp.broadcast_to(em, (BC, N, 1)).reshape(M, 1)
        out2 = out2 + h_new * em2
        return h_new, out2

    z = jnp.zeros((M, D), dtype=jnp.float32)
    _, out2 = jax.lax.fori_loop(0, _S, step_fn, (z, z), unroll=4)
    out_ref[...] = out2.reshape(BC, N, D)


def kernel(obs_emb, observed_mask, lengths, avg_interval, var_plm_rep_tensor,
           rarity_W, W_u, b_u, W_r, b_r, W_c, b_c, ps_W1, ps_b1, ps_W2, ps_b2,
           pg_W1, pg_b1, pg_W2, pg_b2):
    B, S, N, D = obs_emb.shape
    QD = W_u.shape[0]
    f32 = jnp.float32

    def big(W):
        W_oh = jnp.concatenate([W[:, :D, :], W[:, D + 1:, :]], axis=1)
        return W_oh.reshape(QD * 2 * D, D)

    wbig_ru = jnp.concatenate([big(W_r), big(W_u)], axis=1)
    wbig_c = big(W_c)
    def rarrow(W):
        return jnp.pad(W[:, D, :], ((0, _QDP - QD), (0, 0)))
    wrar_ru = jnp.concatenate([rarrow(W_r), rarrow(W_u)], axis=1)
    wrar_c = rarrow(W_c)
    bcat = jnp.concatenate([b_r, b_u, b_c], axis=1)
    bcat = jnp.pad(bcat, ((0, _QDP - QD), (0, 0)))
    psW2 = jnp.pad(ps_W2, ((0, 0), (0, _QDP - QD)))
    psb2 = jnp.pad(ps_b2, (0, _QDP - QD)).reshape(1, _QDP)
    psb1 = ps_b1.reshape(1, -1)
    pgb1 = pg_b1.reshape(1, -1)
    pgb2 = pg_b2.reshape(1, -1)
    maskf = observed_mask.astype(f32)

    adjN, vv, beff, emask = pl.pallas_call(
        _precompute_body,
        out_shape=[
            jax.ShapeDtypeStruct((N, N), f32),
            jax.ShapeDtypeStruct((N, _QDP), f32),
            jax.ShapeDtypeStruct((N, 3 * D), f32),
            jax.ShapeDtypeStruct((B, S), f32),
        ],
    )(var_plm_rep_tensor, ps_W1, psb1, psW2, psb2,
      pg_W1, pgb1, pg_W2, pgb2, bcat, lengths)

    BC = _BC
    rep = lambda *shape: pl.BlockSpec(shape, lambda b: (0,) * len(shape))
    out_call = pl.pallas_call(
        _recurrence_body,
        grid=(B // BC,),
        in_specs=[
            pl.BlockSpec((BC, S, N, D), lambda b: (b, 0, 0, 0)),
            pl.BlockSpec((BC, S, N), lambda b: (b, 0, 0)),
            pl.BlockSpec((BC, S, N), lambda b: (b, 0, 0)),
            pl.BlockSpec((BC, S, 1), lambda b: (b, 0, 0)),
            rep(N, N),
            rep(N, N),
            rep(N, _QDP),
            rep(QD * 2 * D, 2 * D),
            rep(_QDP, 2 * D),
            rep(QD * 2 * D, D),
            rep(_QDP, D),
            rep(N, 3 * D),
        ],
        out_specs=pl.BlockSpec((BC, N, D), lambda b: (b, 0, 0)),
        out_shape=jax.ShapeDtypeStruct((B, N, D), f32),
        compiler_params=pltpu.CompilerParams(
            dimension_semantics=("parallel",),
            vmem_limit_bytes=100 * 1024 * 1024),
    )
    out = out_call(obs_emb, maskf, avg_interval, emask.reshape(B, S, 1),
                   adjN, rarity_W, vv, wbig_ru, wrar_ru, wbig_c, wrar_c, beff)
    return out
